# baseline (speedup 1.0000x reference)
import functools

import jax
import jax.numpy as jnp
from jax import lax
from jax.experimental import pallas as pl
from jax.experimental.pallas import tpu as pltpu
from jax.experimental.pallas import tpu_sc as plsc

Q_EXP = 0.7
K_TRUNC = 0.5
KQ = K_TRUNC ** Q_EXP


@functools.cache
def _make_sc_gather(num_idx: int):
    info = plsc.get_sparse_core_info()
    nw = info.num_cores * info.num_subcores
    assert num_idx % (8 * nw) == 0
    bpw = num_idx // nw

    mesh = plsc.VectorSubcoreMesh(core_axis_name="c", subcore_axis_name="s")

    @functools.partial(
        pl.kernel,
        out_type=jax.ShapeDtypeStruct((num_idx,), jnp.float32),
        mesh=mesh,
        scratch_types=[
            pltpu.VMEM((bpw,), jnp.int32),
            pltpu.VMEM((bpw,), jnp.float32),
            pltpu.SemaphoreType.DMA,
        ],
    )
    def gather_w(weight_hbm, idx_hbm, out_hbm, idx_v, val_v, sem):
        wid = lax.axis_index("s") * info.num_cores + lax.axis_index("c")
        base = wid * bpw
        pltpu.sync_copy(idx_hbm.at[pl.ds(base, bpw)], idx_v)
        pltpu.async_copy(weight_hbm.at[idx_v], val_v, sem).wait()
        pltpu.sync_copy(val_v, out_hbm.at[pl.ds(base, bpw)])

    return gather_w


def _loss_body(scale, logits_ref, tgt_ref, w_ref, out_ref):
    i = pl.program_id(0)
    x = logits_ref[...]
    t = tgt_ref[...]
    w = w_ref[...]
    m = jnp.max(x, axis=1, keepdims=True)
    s = jnp.sum(jnp.exp(x - m), axis=1, keepdims=True)
    col = lax.broadcasted_iota(jnp.int32, x.shape, 1)
    tv = jnp.sum(jnp.where(col == t, x, 0.0), axis=1, keepdims=True)
    ygq = jnp.exp(Q_EXP * (tv - m - jnp.log(s)))
    part = jnp.sum(w * (KQ - ygq), axis=(0, 1), keepdims=True) * scale

    @pl.when(i == 0)
    def _():
        out_ref[...] = jnp.zeros_like(part)

    out_ref[...] += part


def kernel(logits, targets, indexes, weight):
    n, c = logits.shape
    w = _make_sc_gather(n)(weight.reshape(-1), indexes.astype(jnp.int32))

    rows = 512
    nb = n // rows
    scale = 1.0 / (n * Q_EXP)

    acc = pl.pallas_call(
        functools.partial(_loss_body, scale),
        grid=(nb,),
        in_specs=[
            pl.BlockSpec((rows, c), lambda i: (i, 0)),
            pl.BlockSpec((rows, 1), lambda i: (i, 0)),
            pl.BlockSpec((rows, 1), lambda i: (i, 0)),
        ],
        out_specs=pl.BlockSpec((1, 1), lambda i: (0, 0)),
        out_shape=jax.ShapeDtypeStruct((1, 1), jnp.float32),
    )(logits, targets.astype(jnp.int32).reshape(n, 1), w.reshape(n, 1))

    return acc[0, 0]
